# baseline (speedup 1.0000x reference)
import functools

import jax
import jax.numpy as jnp
from jax import lax
from jax.experimental import pallas as pl
from jax.experimental.pallas import tpu as pltpu
from jax.experimental.pallas import tpu_sc as plsc

VOCAB = 100000
N_EMBD = 1024
B = 4
T = 2048
NC, NS, L = 2, 16, 16
NW = NC * NS
T_PER_W = T // NW
C = 16
NCHUNK = T_PER_W // C
NUNIT = NCHUNK * B
NBUF = 3
NOB = 2
NWB = 2
VPU = N_EMBD // L

_mesh = plsc.VectorSubcoreMesh(core_axis_name="c", subcore_axis_name="s")


@functools.partial(
    pl.kernel,
    mesh=_mesh,
    out_type=jax.ShapeDtypeStruct((B * T, N_EMBD), jnp.float32),
    scratch_types=[
        pltpu.VMEM((B * T_PER_W,), jnp.int32),
        [pltpu.VMEM((C, N_EMBD), jnp.float32) for _ in range(NWB)],
        [pltpu.VMEM((C, N_EMBD), jnp.float32) for _ in range(NBUF)],
        [pltpu.VMEM((C, N_EMBD), jnp.float32) for _ in range(NOB)],
        [pltpu.SemaphoreType.DMA for _ in range(NWB)],
        [pltpu.SemaphoreType.DMA for _ in range(NBUF)],
        [pltpu.SemaphoreType.DMA for _ in range(NOB)],
        pltpu.SemaphoreType.DMA,
    ],
)
def _embed(idx_hbm, wte_hbm, wpe_hbm, out_hbm,
           idx_v, wbufs, gbufs, obufs, wsems, gsems, osems, isem):
    wid = lax.axis_index("s") * NC + lax.axis_index("c")
    t0 = pl.multiple_of(wid * T_PER_W, T_PER_W)

    icopies = []
    for b in range(B):
        icopies.append(pltpu.async_copy(
            idx_hbm.at[pl.ds(b * T + t0, T_PER_W)],
            idx_v.at[pl.ds(b * T_PER_W, T_PER_W)], isem))
    for cp in icopies:
        cp.wait()

    gcopies = [None] * NBUF
    ocopies = [None] * NOB
    wcopies = [None] * NWB

    def fire_wpe(c):
        kw = c % NWB
        wcopies[kw] = pltpu.async_copy(
            wpe_hbm.at[pl.ds(t0 + c * C, C)], wbufs[kw], wsems[kw])

    def fire_gather(u):
        c, b = divmod(u, B)
        k = u % NBUF
        gcopies[k] = pltpu.async_copy(
            wte_hbm.at[idx_v.at[pl.ds(b * T_PER_W + c * C, C)]],
            gbufs[k], gsems[k])

    for c in range(min(NWB, NCHUNK)):
        fire_wpe(c)
    for k in range(NBUF):
        fire_gather(k)

    for u in range(NUNIT):
        c, b = divmod(u, B)
        k = u % NBUF
        ko = u % NOB
        kw = c % NWB
        if b == 0:
            wcopies[kw].wait()
        gcopies[k].wait()
        gbuf = gbufs[k]
        obuf = obufs[ko]
        wbuf = wbufs[kw]
        if u >= NOB:
            ocopies[ko].wait()

        def add_row(r, carry):
            def add_half(jg, inner):
                goff = pl.multiple_of(jg * (16 * L), 16 * L)
                for jj in range(16):
                    off = goff + jj * L
                    obuf[r, pl.ds(off, L)] = (
                        gbuf[r, pl.ds(off, L)] + wbuf[r, pl.ds(off, L)])
                return inner

            return lax.fori_loop(0, 4, add_half, carry)

        lax.fori_loop(0, C, add_row, 0)

        base = pl.multiple_of(b * T + t0 + c * C, C)
        ocopies[ko] = pltpu.async_copy(obuf, out_hbm.at[pl.ds(base, C)],
                                       osems[ko])

        nu = u + NBUF
        if nu < NUNIT:
            fire_gather(nu)
        if b == B - 1 and c + NWB < NCHUNK:
            fire_wpe(c + NWB)

    for ko in range(NOB):
        ocopies[ko].wait()


def kernel(idx_cpu, wte, wpe):
    bsz, t = idx_cpu.shape
    idx_flat = idx_cpu.reshape(-1).astype(jnp.int32)
    out = _embed(idx_flat, wte, wpe)
    return out.reshape(bsz, t, N_EMBD)


# speedup vs baseline: 2.3057x; 2.3057x over previous
import functools

import jax
import jax.numpy as jnp
from jax import lax
from jax.experimental import pallas as pl
from jax.experimental.pallas import tpu as pltpu
from jax.experimental.pallas import tpu_sc as plsc

VOCAB = 100000
N_EMBD = 1024
B = 4
T = 2048
NC, NS, L = 2, 16, 16
NW = NC * NS
T_PER_W = T // NW
C = 16
NCHUNK = T_PER_W // C
NUNIT = NCHUNK * B
NBUF = 3
NOB = 2
NWB = 2
VPU = N_EMBD // L

_mesh = plsc.VectorSubcoreMesh(core_axis_name="c", subcore_axis_name="s")


@functools.partial(
    pl.kernel,
    mesh=_mesh,
    out_type=jax.ShapeDtypeStruct((B * T, N_EMBD), jnp.float32),
    scratch_types=[
        pltpu.VMEM((B * T_PER_W,), jnp.int32),
        [pltpu.VMEM((C, N_EMBD), jnp.float32) for _ in range(NWB)],
        [pltpu.VMEM((C, N_EMBD), jnp.float32) for _ in range(NBUF)],
        [pltpu.VMEM((C, N_EMBD), jnp.float32) for _ in range(NOB)],
        [pltpu.SemaphoreType.DMA for _ in range(NWB)],
        [pltpu.SemaphoreType.DMA for _ in range(NBUF)],
        [pltpu.SemaphoreType.DMA for _ in range(NOB)],
        pltpu.SemaphoreType.DMA,
    ],
)
def _embed(idx_hbm, wte_hbm, wpe_hbm, out_hbm,
           idx_v, wbufs, gbufs, obufs, wsems, gsems, osems, isem):
    wid = lax.axis_index("s") * NC + lax.axis_index("c")
    t0 = pl.multiple_of(wid * T_PER_W, T_PER_W)

    icopies = []
    for b in range(B):
        icopies.append(pltpu.async_copy(
            idx_hbm.at[pl.ds(b * T + t0, T_PER_W)],
            idx_v.at[pl.ds(b * T_PER_W, T_PER_W)], isem))
    for cp in icopies:
        cp.wait()

    gcopies = [None] * NBUF
    ocopies = [None] * NOB
    wcopies = [None] * NWB

    def fire_wpe(c):
        kw = c % NWB
        wcopies[kw] = pltpu.async_copy(
            wpe_hbm.at[pl.ds(t0 + c * C, C)], wbufs[kw], wsems[kw])

    H = C // 2

    def fire_gather(u):
        c, b = divmod(u, B)
        k = u % NBUF
        i0 = b * T_PER_W + c * C
        gcopies[k] = (
            pltpu.async_copy(
                wte_hbm.at[idx_v.at[pl.ds(i0, H)]],
                gbufs[k].at[pl.ds(0, H)], gsems[k]),
            pltpu.async_copy(
                wte_hbm.at[idx_v.at[pl.ds(i0 + H, H)]],
                gbufs[k].at[pl.ds(H, H)], gsems[k]),
        )

    for c in range(min(NWB, NCHUNK)):
        fire_wpe(c)
    for k in range(NBUF):
        fire_gather(k)

    for u in range(NUNIT):
        c, b = divmod(u, B)
        k = u % NBUF
        ko = u % NOB
        kw = c % NWB
        if b == 0:
            wcopies[kw].wait()
        gcopies[k][0].wait()
        gcopies[k][1].wait()
        gbuf = gbufs[k]
        obuf = obufs[ko]
        wbuf = wbufs[kw]
        if u >= NOB:
            ocopies[ko].wait()

        def add_row(r, carry):
            def add_half(jg, inner):
                goff = pl.multiple_of(jg * (32 * L), 32 * L)
                for jj in range(32):
                    off = goff + jj * L
                    obuf[r, pl.ds(off, L)] = (
                        gbuf[r, pl.ds(off, L)] + wbuf[r, pl.ds(off, L)])
                return inner

            return lax.fori_loop(0, 2, add_half, carry)

        lax.fori_loop(0, C, add_row, 0)

        base = pl.multiple_of(b * T + t0 + c * C, C)
        ocopies[ko] = pltpu.async_copy(obuf, out_hbm.at[pl.ds(base, C)],
                                       osems[ko])

        nu = u + NBUF
        if nu < NUNIT:
            fire_gather(nu)
        if b == B - 1 and c + NWB < NCHUNK:
            fire_wpe(c + NWB)

    for ko in range(NOB):
        ocopies[ko].wait()


def kernel(idx_cpu, wte, wpe):
    bsz, t = idx_cpu.shape
    idx_flat = idx_cpu.reshape(-1).astype(jnp.int32)
    out = _embed(idx_flat, wte, wpe)
    return out.reshape(bsz, t, N_EMBD)
